# baseline (speedup 1.0000x reference)
import functools

import jax
import jax.numpy as jnp
from jax.experimental import pallas as pl
from jax.experimental.pallas import tpu as pltpu

_B, _N, _C_IN, _C_OUT, _K = 4, 2048, 128, 128, 16
_D = _C_IN + 3
_BR = 512


def _pcdconv_kernel(x_ref, wr_ref, br_ref, wo_ref, out_ref):
    r = pl.program_id(1)
    x_all = x_ref[0]
    x_rows = x_ref[0, pl.ds(r * _BR, _BR), :]

    sq_all = jnp.sum(x_all * x_all, axis=1)
    sq_rows = jnp.sum(x_rows * x_rows, axis=1)
    g = jax.lax.dot_general(
        x_rows, x_all, (((1,), (1,)), ((), ())),
        preferred_element_type=jnp.float32,
        precision=jax.lax.Precision.DEFAULT)
    dist = sq_rows[:, None] + sq_all[None, :] - 2.0 * g

    gi = jax.lax.broadcasted_iota(jnp.int32, (_BR, _N), 0) + r * _BR
    gj = jax.lax.broadcasted_iota(jnp.int32, (_BR, _N), 1)
    dist = jnp.where(gi == gj, jnp.inf, dist)

    w = dist
    for _ in range(_K - 1):
        m = jnp.min(w, axis=1, keepdims=True)
        w = jnp.where(w <= m, jnp.inf, w)
    thresh = jnp.min(w, axis=1, keepdims=True)

    mask = (dist <= thresh).astype(jnp.float32)

    y = jax.lax.dot_general(
        x_all, wr_ref[...], (((1,), (1,)), ((), ())),
        preferred_element_type=jnp.float32,
        precision=jax.lax.Precision.HIGHEST)
    agg = jax.lax.dot_general(
        mask, y, (((1,), (0,)), ((), ())),
        preferred_element_type=jnp.float32,
        precision=jax.lax.Precision.HIGHEST)
    root = jax.lax.dot_general(
        x_rows, wo_ref[...], (((1,), (1,)), ((), ())),
        preferred_element_type=jnp.float32,
        precision=jax.lax.Precision.HIGHEST)

    out_ref[0] = jax.nn.relu(agg + br_ref[...] + root)


@functools.partial(jax.jit, static_argnames=("interpret",))
def _run(xf, W_rel, b_rel, W_root, interpret=False):
    grid = (_B, _N // _BR)
    return pl.pallas_call(
        _pcdconv_kernel,
        grid=grid,
        in_specs=[
            pl.BlockSpec((1, _N, _D), lambda b, r: (b, 0, 0)),
            pl.BlockSpec((_C_OUT, _D), lambda b, r: (0, 0)),
            pl.BlockSpec((1, _C_OUT), lambda b, r: (0, 0)),
            pl.BlockSpec((_C_OUT, _D), lambda b, r: (0, 0)),
        ],
        out_specs=pl.BlockSpec((1, _BR, _C_OUT), lambda b, r: (b, r, 0)),
        out_shape=jax.ShapeDtypeStruct((_B, _N, _C_OUT), jnp.float32),
        compiler_params=pltpu.CompilerParams(
            dimension_semantics=("parallel", "arbitrary")),
        interpret=interpret,
    )(xf, W_rel, b_rel, W_root)


def kernel(x_loc, x_feat, W_rel, b_rel, W_root, interpret=False):
    xf = jnp.concatenate([x_loc, x_feat], axis=1)
    xf = jnp.transpose(xf, (0, 2, 1))
    out = _run(xf, W_rel, b_rel.reshape(1, _C_OUT), W_root, interpret)
    return (x_loc, jnp.transpose(out, (0, 2, 1)))


# speedup vs baseline: 1.4334x; 1.4334x over previous
import functools

import jax
import jax.numpy as jnp
from jax.experimental import pallas as pl
from jax.experimental.pallas import tpu as pltpu

_B, _N, _C_IN, _C_OUT, _K = 4, 2048, 128, 128, 16
_D = _C_IN + 3
_BR = 512


def _pcdconv_kernel(x_ref, wr_ref, br_ref, wo_ref, out_ref,
                    yhi_scr, ylo_scr, sqrow_scr):
    r = pl.program_id(1)
    x_all = x_ref[0]
    x_rows = x_ref[0, pl.ds(r * _BR, _BR), :]

    @pl.when(r == 0)
    def _per_batch():
        y = jax.lax.dot_general(
            x_all, wr_ref[...], (((1,), (1,)), ((), ())),
            preferred_element_type=jnp.float32,
            precision=jax.lax.Precision.HIGHEST)
        y_hi = y.astype(jnp.bfloat16)
        yhi_scr[...] = y_hi
        ylo_scr[...] = (y - y_hi.astype(jnp.float32)).astype(jnp.bfloat16)
        sq = jnp.sum(x_all * x_all, axis=1)
        sqrow_scr[...] = sq[None, :]

    sq_rows = jnp.sum(x_rows * x_rows, axis=1)
    g = jax.lax.dot_general(
        x_rows, x_all, (((1,), (1,)), ((), ())),
        preferred_element_type=jnp.float32,
        precision=jax.lax.Precision.DEFAULT)
    dist = sq_rows[:, None] + sqrow_scr[...] - 2.0 * g

    gi = jax.lax.broadcasted_iota(jnp.int32, (_BR, _N), 0) + r * _BR
    gj = jax.lax.broadcasted_iota(jnp.int32, (_BR, _N), 1)
    dist = jnp.where(gi == gj, jnp.inf, dist)

    w = dist
    for _ in range(_K - 1):
        m = jnp.min(w, axis=1, keepdims=True)
        w = jnp.where(w <= m, jnp.inf, w)
    thresh = jnp.min(w, axis=1, keepdims=True)

    mask = (dist <= thresh).astype(jnp.bfloat16)

    agg = jax.lax.dot_general(
        mask, yhi_scr[...], (((1,), (0,)), ((), ())),
        preferred_element_type=jnp.float32)
    agg += jax.lax.dot_general(
        mask, ylo_scr[...], (((1,), (0,)), ((), ())),
        preferred_element_type=jnp.float32)
    root = jax.lax.dot_general(
        x_rows, wo_ref[...], (((1,), (1,)), ((), ())),
        preferred_element_type=jnp.float32,
        precision=jax.lax.Precision.HIGHEST)

    out_ref[0] = jax.nn.relu(agg + br_ref[...] + root)


@functools.partial(jax.jit, static_argnames=("interpret",))
def _run(xf, W_rel, b_rel, W_root, interpret=False):
    grid = (_B, _N // _BR)
    return pl.pallas_call(
        _pcdconv_kernel,
        grid=grid,
        in_specs=[
            pl.BlockSpec((1, _N, _D), lambda b, r: (b, 0, 0)),
            pl.BlockSpec((_C_OUT, _D), lambda b, r: (0, 0)),
            pl.BlockSpec((1, _C_OUT), lambda b, r: (0, 0)),
            pl.BlockSpec((_C_OUT, _D), lambda b, r: (0, 0)),
        ],
        out_specs=pl.BlockSpec((1, _BR, _C_OUT), lambda b, r: (b, r, 0)),
        out_shape=jax.ShapeDtypeStruct((_B, _N, _C_OUT), jnp.float32),
        scratch_shapes=[
            pltpu.VMEM((_N, _C_OUT), jnp.bfloat16),
            pltpu.VMEM((_N, _C_OUT), jnp.bfloat16),
            pltpu.VMEM((1, _N), jnp.float32),
        ],
        compiler_params=pltpu.CompilerParams(
            dimension_semantics=("parallel", "arbitrary")),
        interpret=interpret,
    )(xf, W_rel, b_rel, W_root)


def kernel(x_loc, x_feat, W_rel, b_rel, W_root, interpret=False):
    xf = jnp.concatenate([x_loc, x_feat], axis=1)
    xf = jnp.transpose(xf, (0, 2, 1))
    out = _run(xf, W_rel, b_rel.reshape(1, _C_OUT), W_root, interpret)
    return (x_loc, jnp.transpose(out, (0, 2, 1)))


# speedup vs baseline: 1.4728x; 1.0275x over previous
import functools

import jax
import jax.numpy as jnp
from jax.experimental import pallas as pl
from jax.experimental.pallas import tpu as pltpu

_B, _N, _C_IN, _C_OUT, _K = 4, 2048, 128, 128, 16
_D = _C_IN + 3
_BR = 1024


def _pcdconv_kernel(x_ref, wr_ref, br_ref, wo_ref, out_ref,
                    yhi_scr, ylo_scr, sqrow_scr):
    r = pl.program_id(1)
    x_all = x_ref[0]
    x_rows = x_ref[0, pl.ds(r * _BR, _BR), :]

    @pl.when(r == 0)
    def _per_batch():
        y = jax.lax.dot_general(
            x_all, wr_ref[...], (((1,), (1,)), ((), ())),
            preferred_element_type=jnp.float32,
            precision=jax.lax.Precision.HIGHEST)
        y_hi = y.astype(jnp.bfloat16)
        yhi_scr[...] = y_hi
        ylo_scr[...] = (y - y_hi.astype(jnp.float32)).astype(jnp.bfloat16)
        sq = jnp.sum(x_all * x_all, axis=1)
        sqrow_scr[...] = sq[None, :]

    sq_rows = jnp.sum(x_rows * x_rows, axis=1)
    g = jax.lax.dot_general(
        x_rows, x_all, (((1,), (1,)), ((), ())),
        preferred_element_type=jnp.float32,
        precision=jax.lax.Precision.DEFAULT)
    dist = sq_rows[:, None] + sqrow_scr[...] - 2.0 * g

    gi = jax.lax.broadcasted_iota(jnp.int32, (_BR, _N), 0) + r * _BR
    gj = jax.lax.broadcasted_iota(jnp.int32, (_BR, _N), 1)
    dist = jnp.where(gi == gj, jnp.inf, dist)

    w = dist
    for _ in range(_K - 1):
        m = jnp.min(w, axis=1, keepdims=True)
        w = jnp.where(w <= m, jnp.inf, w)
    thresh = jnp.min(w, axis=1, keepdims=True)

    mask = (dist <= thresh).astype(jnp.bfloat16)

    agg = jax.lax.dot_general(
        mask, yhi_scr[...], (((1,), (0,)), ((), ())),
        preferred_element_type=jnp.float32)
    agg += jax.lax.dot_general(
        mask, ylo_scr[...], (((1,), (0,)), ((), ())),
        preferred_element_type=jnp.float32)
    root = jax.lax.dot_general(
        x_rows, wo_ref[...], (((1,), (1,)), ((), ())),
        preferred_element_type=jnp.float32,
        precision=jax.lax.Precision.HIGHEST)

    out_ref[0] = jax.nn.relu(agg + br_ref[...] + root)


@functools.partial(jax.jit, static_argnames=("interpret",))
def _run(xf, W_rel, b_rel, W_root, interpret=False):
    grid = (_B, _N // _BR)
    return pl.pallas_call(
        _pcdconv_kernel,
        grid=grid,
        in_specs=[
            pl.BlockSpec((1, _N, _D), lambda b, r: (b, 0, 0)),
            pl.BlockSpec((_C_OUT, _D), lambda b, r: (0, 0)),
            pl.BlockSpec((1, _C_OUT), lambda b, r: (0, 0)),
            pl.BlockSpec((_C_OUT, _D), lambda b, r: (0, 0)),
        ],
        out_specs=pl.BlockSpec((1, _BR, _C_OUT), lambda b, r: (b, r, 0)),
        out_shape=jax.ShapeDtypeStruct((_B, _N, _C_OUT), jnp.float32),
        scratch_shapes=[
            pltpu.VMEM((_N, _C_OUT), jnp.bfloat16),
            pltpu.VMEM((_N, _C_OUT), jnp.bfloat16),
            pltpu.VMEM((1, _N), jnp.float32),
        ],
        compiler_params=pltpu.CompilerParams(
            dimension_semantics=("parallel", "arbitrary")),
        interpret=interpret,
    )(xf, W_rel, b_rel, W_root)


def kernel(x_loc, x_feat, W_rel, b_rel, W_root, interpret=False):
    xf = jnp.concatenate([x_loc, x_feat], axis=1)
    xf = jnp.transpose(xf, (0, 2, 1))
    out = _run(xf, W_rel, b_rel.reshape(1, _C_OUT), W_root, interpret)
    return (x_loc, jnp.transpose(out, (0, 2, 1)))
